# baseline (speedup 1.0000x reference)
import jax
import jax.numpy as jnp
from jax.experimental import pallas as pl
from jax.experimental.pallas import tpu as pltpu

_L = 8
_T = 64
_KT = 8
_D = 1024
_H = 16
_DH = 64


def _kv_body(x_ref, g_ref, b_ref, w_ref, bias_ref, out_ref):
    x = x_ref[0].reshape(_T * _KT, _D)
    m = jnp.mean(x, axis=-1, keepdims=True)
    v = jnp.mean((x - m) ** 2, axis=-1, keepdims=True)
    nx = (x - m) * jax.lax.rsqrt(v + 1e-5) * g_ref[0] + b_ref[0]
    y = jnp.dot(nx, w_ref[0], preferred_element_type=jnp.float32)
    y = y + bias_ref[0, 0, 0]
    for h in range(_H):
        out_ref[0, 0, :, h] = y[:, h * _DH:(h + 1) * _DH].reshape(_T, _KT, _DH)


def kernel(trk_id, adaptor, gates, ln1_g, ln1_b, ln2_g, ln2_b, Wqkv, bqkv, Wo, bo, ls1, ls2, W1, b1, W2, b2):
    g3 = ln1_g.reshape(_L, 1, _D)
    b3 = ln1_b.reshape(_L, 1, _D)
    bq4 = bqkv.reshape(_L, 3, 1, _D)
    return pl.pallas_call(
        _kv_body,
        grid=(_L, 2),
        in_specs=[
            pl.BlockSpec((1, _T, _KT, _D), lambda l, j: (l, 0, 0, 0)),
            pl.BlockSpec((1, 1, _D), lambda l, j: (l, 0, 0)),
            pl.BlockSpec((1, 1, _D), lambda l, j: (l, 0, 0)),
            pl.BlockSpec((1, _D, _D), lambda l, j: (l, 0, 1 + j)),
            pl.BlockSpec((1, 1, 1, _D), lambda l, j: (l, 1 + j, 0, 0)),
        ],
        out_specs=pl.BlockSpec((1, 1, _T, _H, _KT, _DH),
                               lambda l, j: (j, l, 0, 0, 0, 0)),
        out_shape=jax.ShapeDtypeStruct((2, _L, _T, _H, _KT, _DH), jnp.float32),
    )(adaptor, g3, b3, Wqkv, bq4)


# speedup vs baseline: 1.1713x; 1.1713x over previous
import jax
import jax.numpy as jnp
from jax.experimental import pallas as pl
from jax.experimental.pallas import tpu as pltpu

_L = 8
_T = 64
_KT = 8
_D = 1024
_H = 16
_DH = 64


def _kv_body(x_ref, g_ref, b_ref, wk_ref, wv_ref, bias_ref, out_ref):
    x = x_ref[0].reshape(_T * _KT, _D)
    m = jnp.mean(x, axis=-1, keepdims=True)
    v = jnp.mean((x - m) ** 2, axis=-1, keepdims=True)
    nx = (x - m) * jax.lax.rsqrt(v + 1e-5) * g_ref[0] + b_ref[0]
    yk = jnp.dot(nx, wk_ref[0], preferred_element_type=jnp.float32)
    yk = yk + bias_ref[0, 1, 0]
    yv = jnp.dot(nx, wv_ref[0], preferred_element_type=jnp.float32)
    yv = yv + bias_ref[0, 2, 0]
    for h in range(_H):
        sl = slice(h * _DH, (h + 1) * _DH)
        out_ref[0, 0, :, h] = yk[:, sl].reshape(_T, _KT, _DH)
        out_ref[1, 0, :, h] = yv[:, sl].reshape(_T, _KT, _DH)


def kernel(trk_id, adaptor, gates, ln1_g, ln1_b, ln2_g, ln2_b, Wqkv, bqkv, Wo, bo, ls1, ls2, W1, b1, W2, b2):
    g3 = ln1_g.reshape(_L, 1, _D)
    b3 = ln1_b.reshape(_L, 1, _D)
    bq4 = bqkv.reshape(_L, 3, 1, _D)
    return pl.pallas_call(
        _kv_body,
        grid=(_L,),
        in_specs=[
            pl.BlockSpec((1, _T, _KT, _D), lambda l: (l, 0, 0, 0)),
            pl.BlockSpec((1, 1, _D), lambda l: (l, 0, 0)),
            pl.BlockSpec((1, 1, _D), lambda l: (l, 0, 0)),
            pl.BlockSpec((1, _D, _D), lambda l: (l, 0, 1)),
            pl.BlockSpec((1, _D, _D), lambda l: (l, 0, 2)),
            pl.BlockSpec((1, 3, 1, _D), lambda l: (l, 0, 0, 0)),
        ],
        out_specs=pl.BlockSpec((2, 1, _T, _H, _KT, _DH), lambda l: (0, l, 0, 0, 0, 0)),
        out_shape=jax.ShapeDtypeStruct((2, _L, _T, _H, _KT, _DH), jnp.float32),
        compiler_params=pltpu.CompilerParams(dimension_semantics=("parallel",)),
    )(adaptor, g3, b3, Wqkv, Wqkv, bq4)
